# baseline (speedup 1.0000x reference)
import jax
import jax.numpy as jnp
from jax.experimental import pallas as pl
from jax.experimental.pallas import tpu as pltpu

_K = 32
_BISECT_ITERS = 25


def _normalized(x):
    nrm = jnp.sqrt(jnp.sum(x * x, axis=1, keepdims=True))
    return x / jnp.maximum(nrm, 1e-12)


def _sim(xn):
    return jax.lax.dot_general(
        xn, xn, (((1,), (1,)), ((), ())), preferred_element_type=jnp.float32
    )


def _threshold_body(x_ref, t_ref, s_ref):
    xn = _normalized(x_ref[0])
    n = xn.shape[0]
    s_ref[...] = _sim(xn)

    lo = jnp.full((n, 1), -1.0, jnp.float32)
    hi = jnp.full((n, 1), 1.0, jnp.float32)
    ch = 512
    ones_col = jnp.ones((ch, 8), jnp.bfloat16)

    def body(_, carry):
        lo, hi = carry
        mid = (lo + hi) * 0.5
        cnt = jnp.zeros((n, 1), jnp.float32)
        for c in range(n // ch):
            ind = (s_ref[:, c * ch:(c + 1) * ch] >= mid).astype(jnp.bfloat16)
            cnt = cnt + jax.lax.dot_general(
                ind, ones_col, (((1,), (0,)), ((), ())),
                preferred_element_type=jnp.float32,
            )[:, :1]
        pred = cnt >= _K
        return jnp.where(pred, mid, lo), jnp.where(pred, hi, mid)

    lo, _ = jax.lax.fori_loop(0, _BISECT_ITERS, body, (lo, hi))
    t_ref[0] = lo.reshape(1, n)


def _mask_body(x_ref, t_ref, o_ref):
    xn = _normalized(x_ref[0])
    n = xn.shape[0]
    s = _sim(xn)
    t_row = t_ref[0]
    t_col = t_row.reshape(n, 1)
    keep = (s >= t_col).astype(jnp.float32) + (s >= t_row).astype(jnp.float32)
    o_ref[0] = s * (keep * 0.5)


def kernel(x):
    b, n, d = x.shape
    t = pl.pallas_call(
        _threshold_body,
        grid=(b,),
        in_specs=[pl.BlockSpec((1, n, d), lambda i: (i, 0, 0))],
        out_specs=pl.BlockSpec((1, 1, n), lambda i: (i, 0, 0)),
        out_shape=jax.ShapeDtypeStruct((b, 1, n), jnp.float32),
        scratch_shapes=[pltpu.VMEM((n, n), jnp.float32)],
        compiler_params=pltpu.CompilerParams(
            dimension_semantics=("arbitrary",),
        ),
    )(x)
    return pl.pallas_call(
        _mask_body,
        grid=(b,),
        in_specs=[
            pl.BlockSpec((1, n, d), lambda i: (i, 0, 0)),
            pl.BlockSpec((1, 1, n), lambda i: (i, 0, 0)),
        ],
        out_specs=pl.BlockSpec((1, n, n), lambda i: (i, 0, 0)),
        out_shape=jax.ShapeDtypeStruct((b, n, n), jnp.float32),
        compiler_params=pltpu.CompilerParams(
            dimension_semantics=("arbitrary",),
        ),
    )(x, t)
